# Initial kernel scaffold; baseline (speedup 1.0000x reference)
import jax
import jax.numpy as jnp
from jax.experimental import pallas as pl


def kernel(h, edge_index, attn_w):
    raise NotImplementedError("write your pallas kernel here")


# speedup vs baseline: 2.2224x; 2.2224x over previous
import jax
import jax.numpy as jnp
from jax import lax
from jax.experimental import pallas as pl
from jax.experimental.pallas import tpu as pltpu
from jax.experimental.pallas import tpu_sc as plsc

_N = 10000
_E = 320000
_D = 128
_NP = 10240
_NT = 16
_ET = _E // _NT
_K = 80
_KP = 128
_EC = 2000
_NC = _ET // _EC
_GC = _EC // _K
_CW = _GC * _KP
_RB = 1280


def _pq_body(h_ref, w_ref, pq_ref, b_ref):
    h = h_ref[...]
    w = w_ref[...]
    pq = lax.dot_general(h, w, (((1,), (1,)), ((), ())),
                         preferred_element_type=jnp.float32)
    pq_ref[...] = pq
    s = jnp.max(pq[:, 0]) + jnp.max(pq[:, 1])
    b = jnp.where(s >= 0, s, 0.01 * s)
    b_ref[...] = jnp.full((8, 128), b, jnp.float32)


def _sc_body(h_hbm, srcg_hbm, dstg_hbm, p_hbm, q_hbm, b_hbm,
             u_hbm, s_hbm,
             src_c, dst_c, ex_c, p_v, q_v, b_v, rows_v, zbuf, szbuf,
             u_sh, s_sh):
    cid = lax.axis_index("c")
    sid = lax.axis_index("s")
    nbase = sid * 640

    zv = jnp.zeros((16,), jnp.float32)

    def zrow(r, c):
        for c8 in range(8):
            zbuf[r, pl.ds(16 * c8, 16)] = zv
        return c
    lax.fori_loop(0, 32, zrow, 0)

    def zs(i, c):
        szbuf[pl.ds(16 * i, 16)] = zv
        return c
    lax.fori_loop(0, 40, zs, 0)

    for k in range(20):
        pltpu.sync_copy(zbuf, u_sh.at[pl.ds(nbase + 32 * k, 32)])
    pltpu.sync_copy(szbuf, s_sh.at[pl.ds(nbase, 640)])

    pltpu.sync_copy(p_hbm, p_v)
    pltpu.sync_copy(q_hbm, q_v)
    pltpu.sync_copy(b_hbm.at[0], b_v)

    plsc.subcore_barrier()

    iota = lax.iota(jnp.int32, 16)
    bvec = b_v[pl.ds(0, 16)]

    def chunk(ci, cc):
        w = sid * _NC + ci
        pltpu.sync_copy(srcg_hbm.at[w, 0], src_c)
        pltpu.sync_copy(dstg_hbm.at[w, 0], dst_c)

        def exg(g, c):
            eb = g * _KP
            for jb in range(5):
                sv = src_c[pl.ds(eb + 16 * jb, 16)]
                dv = dst_c[pl.ds(eb + 16 * jb, 16)]
                pv = plsc.load_gather(p_v, [sv])
                qv = plsc.load_gather(q_v, [dv])
                a = pv + qv
                e = jnp.where(a >= 0, a, a * 0.01)
                ex_c[pl.ds(eb + 16 * jb, 16)] = jnp.exp(e - bvec)
            return c
        lax.fori_loop(0, _GC, exg, 0)

        def heavy(g, c):
            eb = g * _KP
            pltpu.sync_copy(h_hbm.at[src_c.at[pl.ds(eb, _K)]], rows_v)
            exvs = [ex_c[pl.ds(eb + 16 * jb, 16)] for jb in range(5)]
            jvecs = [iota + 16 * jb for jb in range(5)]

            def scale_cb(cb, c2):
                for c8 in range(8):
                    cvec = jnp.full((16,), cb * 8 + c8, jnp.int32)
                    for jb in range(5):
                        x = plsc.load_gather(rows_v, [jvecs[jb], cvec])
                        plsc.store_scatter(rows_v, [jvecs[jb], cvec],
                                           x * exvs[jb])
                return c2
            lax.fori_loop(0, 16, scale_cb, 0)

            pltpu.sync_copy(rows_v, u_sh.at[dst_c.at[pl.ds(eb, _K)]],
                            add=True)
            pltpu.sync_copy(ex_c.at[pl.ds(eb, _K)],
                            s_sh.at[dst_c.at[pl.ds(eb, _K)]], add=True)
            return c
        lax.fori_loop(0, _GC, heavy, 0)
        return cc
    lax.fori_loop(0, _NC, chunk, 0)

    plsc.subcore_barrier()

    for k in range(20):
        pltpu.sync_copy(u_sh.at[pl.ds(nbase + 32 * k, 32)], zbuf)
        pltpu.sync_copy(zbuf, u_hbm.at[cid, pl.ds(nbase + 32 * k, 32)])
    pltpu.sync_copy(s_sh.at[pl.ds(nbase, 640)], szbuf)
    pltpu.sync_copy(szbuf, s_hbm.at[cid, pl.ds(nbase, 640)])


def _div_body(u_ref, st_ref, o_ref):
    num = u_ref[0]
    den = st_ref[:, 0:1]
    o_ref[...] = jnp.where(den > 0, num / den, 0.0)


def _pad_groups(x):
    xg = x.reshape(_NT * _NC, _GC, _K)
    xg = jnp.pad(xg, ((0, 0), (0, 0), (0, _KP - _K)))
    return xg.reshape(_NT * _NC, 1, _CW)


@jax.jit
def kernel(h, edge_index, attn_w):
    w = attn_w[:, 0].reshape(2, _D)

    pq, b = pl.pallas_call(
        _pq_body,
        out_shape=[jax.ShapeDtypeStruct((_N, 2), jnp.float32),
                   jax.ShapeDtypeStruct((8, 128), jnp.float32)],
    )(h, w)

    srcg = _pad_groups(edge_index[0])
    dstg = _pad_groups(edge_index[1])

    mesh = plsc.VectorSubcoreMesh(core_axis_name="c", subcore_axis_name="s",
                                  num_cores=1)
    u2, s2 = pl.kernel(
        _sc_body,
        out_type=[jax.ShapeDtypeStruct((1, _NP, _D), jnp.float32),
                  jax.ShapeDtypeStruct((1, _NP), jnp.float32)],
        mesh=mesh,
        compiler_params=pltpu.CompilerParams(needs_layout_passes=False),
        scratch_types=[
            pltpu.VMEM((_CW,), jnp.int32),
            pltpu.VMEM((_CW,), jnp.int32),
            pltpu.VMEM((_CW,), jnp.float32),
            pltpu.VMEM((_N,), jnp.float32),
            pltpu.VMEM((_N,), jnp.float32),
            pltpu.VMEM((128,), jnp.float32),
            pltpu.VMEM((_K, _D), jnp.float32),
            pltpu.VMEM((32, _D), jnp.float32),
            pltpu.VMEM((640,), jnp.float32),
            pltpu.VMEM_SHARED((_NP, _D), jnp.float32),
            pltpu.VMEM_SHARED((_NP,), jnp.float32),
        ],
    )(h, srcg, dstg, pq[:, 0], pq[:, 1], b)

    st = s2.T
    out = pl.pallas_call(
        _div_body,
        grid=(_NP // _RB,),
        in_specs=[pl.BlockSpec((1, _RB, _D), lambda i: (0, i, 0)),
                  pl.BlockSpec((_RB, 1), lambda i: (i, 0))],
        out_specs=pl.BlockSpec((_RB, _D), lambda i: (i, 0)),
        out_shape=jax.ShapeDtypeStruct((_NP, _D), jnp.float32),
    )(u2, st)
    return out[:_N]


# speedup vs baseline: 12.2733x; 5.5225x over previous
import jax
import jax.numpy as jnp
from jax import lax
from jax.experimental import pallas as pl
from jax.experimental.pallas import tpu as pltpu
from jax.experimental.pallas import tpu_sc as plsc

_N = 10000
_E = 320000
_D = 128
_NP = 10240
_NT = 16
_ET = _E // _NT
_K = 80
_KP = 128
_EC = 2000
_NC = _ET // _EC
_GC = _EC // _K
_CW = _GC * _KP
_RB = 1280


def _pq_body(h_ref, w_ref, pq_ref, b_ref):
    h = h_ref[...]
    w = w_ref[...]
    pq = lax.dot_general(h, w, (((1,), (1,)), ((), ())),
                         preferred_element_type=jnp.float32)
    pq_ref[...] = pq
    s = jnp.max(pq[:, 0]) + jnp.max(pq[:, 1])
    b = jnp.where(s >= 0, s, 0.01 * s)
    b_ref[...] = jnp.full((8, 128), b, jnp.float32)


def _sc_body(h_hbm, srcg_hbm, dstg_hbm, p_hbm, q_hbm, b_hbm,
             u_hbm, s_hbm,
             src_c, dst_c, ex_c, p_v, q_v, b_v, rows_v, zbuf, szbuf,
             u_sh, s_sh):
    cid = lax.axis_index("c")
    sid = lax.axis_index("s")
    nbase = sid * 640

    zv = jnp.zeros((16,), jnp.float32)

    def zrow(r, c):
        for c8 in range(8):
            zbuf[r, pl.ds(16 * c8, 16)] = zv
        return c
    lax.fori_loop(0, 32, zrow, 0)

    def zs(i, c):
        szbuf[pl.ds(16 * i, 16)] = zv
        return c
    lax.fori_loop(0, 40, zs, 0)

    for k in range(20):
        pltpu.sync_copy(zbuf, u_sh.at[pl.ds(nbase + 32 * k, 32)])
    pltpu.sync_copy(szbuf, s_sh.at[pl.ds(nbase, 640)])

    pltpu.sync_copy(p_hbm, p_v)
    pltpu.sync_copy(q_hbm, q_v)
    pltpu.sync_copy(b_hbm.at[0], b_v)

    plsc.subcore_barrier()

    iota = lax.iota(jnp.int32, 16)
    bvec = b_v[pl.ds(0, 16)]

    def chunk(ci, cc):
        w = sid * _NC + ci
        pltpu.sync_copy(srcg_hbm.at[w, 0], src_c)
        pltpu.sync_copy(dstg_hbm.at[w, 0], dst_c)

        def exg(g, c):
            eb = g * _KP
            for jb in range(5):
                sv = src_c[pl.ds(eb + 16 * jb, 16)]
                dv = dst_c[pl.ds(eb + 16 * jb, 16)]
                pv = plsc.load_gather(p_v, [sv])
                qv = plsc.load_gather(q_v, [dv])
                a = pv + qv
                e = jnp.where(a >= 0, a, a * 0.01)
                ex_c[pl.ds(eb + 16 * jb, 16)] = jnp.exp(e - bvec)
            return c
        lax.fori_loop(0, _GC, exg, 0)

        def heavy(g, c):
            eb = g * _KP
            pltpu.sync_copy(h_hbm.at[src_c.at[pl.ds(eb, _K)]], rows_v)

            for jb in range(5):
                exv16 = ex_c[pl.ds(eb + 16 * jb, 16)]

                @plsc.parallel_loop(0, 16, unroll=2)
                def scale_row(i):
                    exv = exv16[jnp.full((16,), i, jnp.int32)]
                    jf = jnp.full((16,), 16 * jb + i, jnp.int32)
                    for c8 in range(8):
                        cvec = iota + 16 * c8
                        x = plsc.load_gather(rows_v, [jf, cvec])
                        plsc.store_scatter(rows_v, [jf, cvec], x * exv)

            pltpu.sync_copy(rows_v, u_sh.at[dst_c.at[pl.ds(eb, _K)]],
                            add=True)
            pltpu.sync_copy(ex_c.at[pl.ds(eb, _K)],
                            s_sh.at[dst_c.at[pl.ds(eb, _K)]], add=True)
            return c
        lax.fori_loop(0, _GC, heavy, 0)
        return cc
    lax.fori_loop(0, _NC, chunk, 0)

    plsc.subcore_barrier()

    for k in range(20):
        pltpu.sync_copy(u_sh.at[pl.ds(nbase + 32 * k, 32)], zbuf)
        pltpu.sync_copy(zbuf, u_hbm.at[cid, pl.ds(nbase + 32 * k, 32)])
    pltpu.sync_copy(s_sh.at[pl.ds(nbase, 640)], szbuf)
    pltpu.sync_copy(szbuf, s_hbm.at[cid, pl.ds(nbase, 640)])


def _div_body(u_ref, st_ref, o_ref):
    num = u_ref[0]
    den = st_ref[:, 0:1]
    o_ref[...] = jnp.where(den > 0, num / den, 0.0)


def _pad_groups(x):
    xg = x.reshape(_NT * _NC, _GC, _K)
    xg = jnp.pad(xg, ((0, 0), (0, 0), (0, _KP - _K)))
    return xg.reshape(_NT * _NC, 1, _CW)


@jax.jit
def kernel(h, edge_index, attn_w):
    w = attn_w[:, 0].reshape(2, _D)

    pq, b = pl.pallas_call(
        _pq_body,
        out_shape=[jax.ShapeDtypeStruct((_N, 2), jnp.float32),
                   jax.ShapeDtypeStruct((8, 128), jnp.float32)],
    )(h, w)

    srcg = _pad_groups(edge_index[0])
    dstg = _pad_groups(edge_index[1])

    mesh = plsc.VectorSubcoreMesh(core_axis_name="c", subcore_axis_name="s",
                                  num_cores=1)
    u2, s2 = pl.kernel(
        _sc_body,
        out_type=[jax.ShapeDtypeStruct((1, _NP, _D), jnp.float32),
                  jax.ShapeDtypeStruct((1, _NP), jnp.float32)],
        mesh=mesh,
        compiler_params=pltpu.CompilerParams(needs_layout_passes=False),
        scratch_types=[
            pltpu.VMEM((_CW,), jnp.int32),
            pltpu.VMEM((_CW,), jnp.int32),
            pltpu.VMEM((_CW,), jnp.float32),
            pltpu.VMEM((_N,), jnp.float32),
            pltpu.VMEM((_N,), jnp.float32),
            pltpu.VMEM((128,), jnp.float32),
            pltpu.VMEM((_K, _D), jnp.float32),
            pltpu.VMEM((32, _D), jnp.float32),
            pltpu.VMEM((640,), jnp.float32),
            pltpu.VMEM_SHARED((_NP, _D), jnp.float32),
            pltpu.VMEM_SHARED((_NP,), jnp.float32),
        ],
    )(h, srcg, dstg, pq[:, 0], pq[:, 1], b)

    st = s2.T
    out = pl.pallas_call(
        _div_body,
        grid=(_NP // _RB,),
        in_specs=[pl.BlockSpec((1, _RB, _D), lambda i: (0, i, 0)),
                  pl.BlockSpec((_RB, 1), lambda i: (i, 0))],
        out_specs=pl.BlockSpec((_RB, _D), lambda i: (i, 0)),
        out_shape=jax.ShapeDtypeStruct((_NP, _D), jnp.float32),
    )(u2, st)
    return out[:_N]


# speedup vs baseline: 21.1659x; 1.7245x over previous
import jax
import jax.numpy as jnp
from jax import lax
from jax.experimental import pallas as pl
from jax.experimental.pallas import tpu as pltpu
from jax.experimental.pallas import tpu_sc as plsc

_N = 10000
_E = 320000
_D = 128
_NP = 10240
_NT = 32
_ET = _E // _NT
_K = 80
_KP = 128
_EC = 2000
_NC = _ET // _EC
_GC = _EC // _K
_CW = _GC * _KP
_RB = 1280


def _pq_body(h_ref, w_ref, pq_ref, b_ref):
    h = h_ref[...]
    w = w_ref[...]
    pq = lax.dot_general(h, w, (((1,), (1,)), ((), ())),
                         preferred_element_type=jnp.float32)
    pq_ref[...] = pq
    s = jnp.max(pq[:, 0]) + jnp.max(pq[:, 1])
    b = jnp.where(s >= 0, s, 0.01 * s)
    b_ref[...] = jnp.full((8, 128), b, jnp.float32)


def _sc_body(h_hbm, srcg_hbm, dstg_hbm, p_hbm, q_hbm, b_hbm,
             u_hbm, s_hbm,
             src_c, dst_c, ex_c, p_v, q_v, b_v, rows_v, zbuf, szbuf,
             u_sh, s_sh):
    cid = lax.axis_index("c")
    sid = lax.axis_index("s")
    wid = cid * 16 + sid
    nbase = sid * 640

    zv = jnp.zeros((16,), jnp.float32)

    def zrow(r, c):
        for c8 in range(8):
            zbuf[r, pl.ds(16 * c8, 16)] = zv
        return c
    lax.fori_loop(0, 32, zrow, 0)

    def zs(i, c):
        szbuf[pl.ds(16 * i, 16)] = zv
        return c
    lax.fori_loop(0, 40, zs, 0)

    for k in range(20):
        pltpu.sync_copy(zbuf, u_sh.at[pl.ds(nbase + 32 * k, 32)])
    pltpu.sync_copy(szbuf, s_sh.at[pl.ds(nbase, 640)])

    pltpu.sync_copy(p_hbm, p_v)
    pltpu.sync_copy(q_hbm, q_v)
    pltpu.sync_copy(b_hbm.at[0], b_v)

    plsc.subcore_barrier()

    iota = lax.iota(jnp.int32, 16)
    bvec = b_v[pl.ds(0, 16)]

    def chunk(ci, cc):
        w = wid * _NC + ci
        pltpu.sync_copy(srcg_hbm.at[w, 0], src_c)
        pltpu.sync_copy(dstg_hbm.at[w, 0], dst_c)

        def exg(g, c):
            eb = g * _KP
            for jb in range(5):
                sv = src_c[pl.ds(eb + 16 * jb, 16)]
                dv = dst_c[pl.ds(eb + 16 * jb, 16)]
                pv = plsc.load_gather(p_v, [sv])
                qv = plsc.load_gather(q_v, [dv])
                a = pv + qv
                e = jnp.where(a >= 0, a, a * 0.01)
                ex_c[pl.ds(eb + 16 * jb, 16)] = jnp.exp(e - bvec)
            return c
        lax.fori_loop(0, _GC, exg, 0)

        def heavy(g, c):
            eb = g * _KP
            pltpu.sync_copy(h_hbm.at[src_c.at[pl.ds(eb, _K)]], rows_v)

            for jb in range(5):
                exv16 = ex_c[pl.ds(eb + 16 * jb, 16)]

                @plsc.parallel_loop(0, 16, unroll=2)
                def scale_row(i):
                    exv = exv16[jnp.full((16,), i, jnp.int32)]
                    jf = jnp.full((16,), 16 * jb + i, jnp.int32)
                    for c8 in range(8):
                        cvec = iota + 16 * c8
                        x = plsc.load_gather(rows_v, [jf, cvec])
                        plsc.store_scatter(rows_v, [jf, cvec], x * exv)

            pltpu.sync_copy(rows_v, u_sh.at[dst_c.at[pl.ds(eb, _K)]],
                            add=True)
            pltpu.sync_copy(ex_c.at[pl.ds(eb, _K)],
                            s_sh.at[dst_c.at[pl.ds(eb, _K)]], add=True)
            return c
        lax.fori_loop(0, _GC, heavy, 0)
        return cc
    lax.fori_loop(0, _NC, chunk, 0)

    plsc.subcore_barrier()

    for k in range(20):
        pltpu.sync_copy(u_sh.at[pl.ds(nbase + 32 * k, 32)], zbuf)
        pltpu.sync_copy(zbuf, u_hbm.at[cid, pl.ds(nbase + 32 * k, 32)])
    pltpu.sync_copy(s_sh.at[pl.ds(nbase, 640)], szbuf)
    pltpu.sync_copy(szbuf, s_hbm.at[cid, pl.ds(nbase, 640)])


def _div_body(u_ref, st_ref, o_ref):
    num = u_ref[0] + u_ref[1]
    st = st_ref[...]
    den = st[:, 0:1] + st[:, 1:2]
    o_ref[...] = jnp.where(den > 0, num / den, 0.0)


def _pad_groups(x):
    xg = x.reshape(_NT * _NC, _GC, _K)
    xg = jnp.pad(xg, ((0, 0), (0, 0), (0, _KP - _K)))
    return xg.reshape(_NT * _NC, 1, _CW)


@jax.jit
def kernel(h, edge_index, attn_w):
    w = attn_w[:, 0].reshape(2, _D)

    pq, b = pl.pallas_call(
        _pq_body,
        out_shape=[jax.ShapeDtypeStruct((_N, 2), jnp.float32),
                   jax.ShapeDtypeStruct((8, 128), jnp.float32)],
    )(h, w)

    srcg = _pad_groups(edge_index[0])
    dstg = _pad_groups(edge_index[1])

    mesh = plsc.VectorSubcoreMesh(core_axis_name="c", subcore_axis_name="s",
                                  num_cores=2)
    u2, s2 = pl.kernel(
        _sc_body,
        out_type=[jax.ShapeDtypeStruct((2, _NP, _D), jnp.float32),
                  jax.ShapeDtypeStruct((2, _NP), jnp.float32)],
        mesh=mesh,
        compiler_params=pltpu.CompilerParams(needs_layout_passes=False),
        scratch_types=[
            pltpu.VMEM((_CW,), jnp.int32),
            pltpu.VMEM((_CW,), jnp.int32),
            pltpu.VMEM((_CW,), jnp.float32),
            pltpu.VMEM((_N,), jnp.float32),
            pltpu.VMEM((_N,), jnp.float32),
            pltpu.VMEM((128,), jnp.float32),
            pltpu.VMEM((_K, _D), jnp.float32),
            pltpu.VMEM((32, _D), jnp.float32),
            pltpu.VMEM((640,), jnp.float32),
            pltpu.VMEM_SHARED((_NP, _D), jnp.float32),
            pltpu.VMEM_SHARED((_NP,), jnp.float32),
        ],
    )(h, srcg, dstg, pq[:, 0], pq[:, 1], b)

    st = s2.T
    out = pl.pallas_call(
        _div_body,
        grid=(_NP // _RB,),
        in_specs=[pl.BlockSpec((2, _RB, _D), lambda i: (0, i, 0)),
                  pl.BlockSpec((_RB, 2), lambda i: (i, 0))],
        out_specs=pl.BlockSpec((_RB, _D), lambda i: (i, 0)),
        out_shape=jax.ShapeDtypeStruct((_NP, _D), jnp.float32),
    )(u2, st)
    return out[:_N]


# speedup vs baseline: 32.7728x; 1.5484x over previous
import jax
import jax.numpy as jnp
from jax import lax
from jax.experimental import pallas as pl
from jax.experimental.pallas import tpu as pltpu
from jax.experimental.pallas import tpu_sc as plsc

_N = 10000
_E = 320000
_D = 128
_NP = 10240
_NT = 32
_ET = _E // _NT
_K = 80
_KP = 80
_EC = 2000
_NC = _ET // _EC
_GC = _EC // _K
_CW = _GC * _KP
_RB = 1280


def _pq_body(h_ref, w_ref, pq_ref, b_ref):
    h = h_ref[...]
    w = w_ref[...]
    pq = lax.dot_general(h, w, (((1,), (1,)), ((), ())),
                         preferred_element_type=jnp.float32)
    pq_ref[...] = pq
    s = jnp.max(pq[:, 0]) + jnp.max(pq[:, 1])
    b = jnp.where(s >= 0, s, 0.01 * s)
    b_ref[...] = jnp.full((8, 128), b, jnp.float32)


def _sc_body(h_hbm, srcg_hbm, dstg_hbm, p_hbm, q_hbm, b_hbm,
             u_hbm, s_hbm,
             src_c, dst_c, ex_c, p_v, q_v, b_v, rows2, szbuf,
             u_sh, s_sh, gsem):
    cid = lax.axis_index("c")
    sid = lax.axis_index("s")
    wid = cid * 16 + sid
    nbase = sid * 640

    zv = jnp.zeros((16,), jnp.float32)

    def zrow(r, c):
        for c8 in range(8):
            rows2[0, r, pl.ds(16 * c8, 16)] = zv
        return c
    lax.fori_loop(0, 80, zrow, 0)

    def zs(i, c):
        szbuf[pl.ds(16 * i, 16)] = zv
        return c
    lax.fori_loop(0, 40, zs, 0)

    for k in range(8):
        pltpu.sync_copy(rows2.at[0], u_sh.at[pl.ds(nbase + 80 * k, 80)])
    pltpu.sync_copy(szbuf, s_sh.at[pl.ds(nbase, 640)])

    pltpu.sync_copy(p_hbm, p_v)
    pltpu.sync_copy(q_hbm, q_v)
    pltpu.sync_copy(b_hbm.at[0], b_v)

    plsc.subcore_barrier()

    iota = lax.iota(jnp.int32, 16)
    bvec = b_v[pl.ds(0, 16)]

    def chunk(ci, cc):
        w = wid * _NC + ci
        pltpu.sync_copy(srcg_hbm.at[w, 0], src_c)
        pltpu.sync_copy(dstg_hbm.at[w, 0], dst_c)

        pltpu.async_copy(h_hbm.at[src_c.at[pl.ds(0, _K)]], rows2.at[0],
                         gsem.at[0])

        def exg(g, c):
            eb = g * _KP
            for jb in range(5):
                sv = src_c[pl.ds(eb + 16 * jb, 16)]
                dv = dst_c[pl.ds(eb + 16 * jb, 16)]
                pv = plsc.load_gather(p_v, [sv])
                qv = plsc.load_gather(q_v, [dv])
                a = pv + qv
                e = jnp.where(a >= 0, a, a * 0.01)
                ex_c[pl.ds(eb + 16 * jb, 16)] = jnp.exp(e - bvec)
            return c
        lax.fori_loop(0, _GC, exg, 0)

        def heavy(g, c):
            eb = g * _KP
            b = g % 2
            bf = jnp.full((16,), b, jnp.int32)

            @pl.when(g + 1 < _GC)
            def _():
                eb1 = (g + 1) * _KP
                pltpu.async_copy(h_hbm.at[src_c.at[pl.ds(eb1, _K)]],
                                 rows2.at[1 - b], gsem.at[1 - b])

            pltpu.make_async_copy(h_hbm.at[src_c.at[pl.ds(eb, _K)]],
                                  rows2.at[b], gsem.at[b]).wait()

            for jb in range(5):
                exv16 = ex_c[pl.ds(eb + 16 * jb, 16)]

                @plsc.parallel_loop(0, 16, unroll=2)
                def scale_row(i):
                    exv = exv16[jnp.full((16,), i, jnp.int32)]
                    jf = jnp.full((16,), 16 * jb + i, jnp.int32)
                    for c8 in range(8):
                        cvec = iota + 16 * c8
                        x = plsc.load_gather(rows2, [bf, jf, cvec])
                        plsc.store_scatter(rows2, [bf, jf, cvec], x * exv)

            pltpu.sync_copy(rows2.at[b], u_sh.at[dst_c.at[pl.ds(eb, _K)]],
                            add=True)
            pltpu.sync_copy(ex_c.at[pl.ds(eb, _K)],
                            s_sh.at[dst_c.at[pl.ds(eb, _K)]], add=True)
            return c
        lax.fori_loop(0, _GC, heavy, 0)
        return cc
    lax.fori_loop(0, _NC, chunk, 0)

    plsc.subcore_barrier()

    for k in range(8):
        pltpu.sync_copy(u_sh.at[pl.ds(nbase + 80 * k, 80)], rows2.at[0])
        pltpu.sync_copy(rows2.at[0], u_hbm.at[cid, pl.ds(nbase + 80 * k, 80)])
    pltpu.sync_copy(s_sh.at[pl.ds(nbase, 640)], szbuf)
    pltpu.sync_copy(szbuf, s_hbm.at[cid, pl.ds(nbase, 640)])


def _div_body(u_ref, st_ref, o_ref):
    num = u_ref[0] + u_ref[1]
    st = st_ref[...]
    den = st[:, 0:1] + st[:, 1:2]
    o_ref[...] = jnp.where(den > 0, num / den, 0.0)


def _pad_groups(x):
    xg = x.reshape(_NT * _NC, _GC, _K)
    xg = jnp.pad(xg, ((0, 0), (0, 0), (0, _KP - _K)))
    return xg.reshape(_NT * _NC, 1, _CW)


@jax.jit
def kernel(h, edge_index, attn_w):
    w = attn_w[:, 0].reshape(2, _D)

    pq, b = pl.pallas_call(
        _pq_body,
        out_shape=[jax.ShapeDtypeStruct((_N, 2), jnp.float32),
                   jax.ShapeDtypeStruct((8, 128), jnp.float32)],
    )(h, w)

    srcg = _pad_groups(edge_index[0])
    dstg = _pad_groups(edge_index[1])

    mesh = plsc.VectorSubcoreMesh(core_axis_name="c", subcore_axis_name="s",
                                  num_cores=2)
    u2, s2 = pl.kernel(
        _sc_body,
        out_type=[jax.ShapeDtypeStruct((2, _NP, _D), jnp.float32),
                  jax.ShapeDtypeStruct((2, _NP), jnp.float32)],
        mesh=mesh,
        compiler_params=pltpu.CompilerParams(needs_layout_passes=False),
        scratch_types=[
            pltpu.VMEM((_CW,), jnp.int32),
            pltpu.VMEM((_CW,), jnp.int32),
            pltpu.VMEM((_CW,), jnp.float32),
            pltpu.VMEM((_N,), jnp.float32),
            pltpu.VMEM((_N,), jnp.float32),
            pltpu.VMEM((128,), jnp.float32),
            pltpu.VMEM((2, _K, _D), jnp.float32),
            pltpu.VMEM((640,), jnp.float32),
            pltpu.VMEM_SHARED((_NP, _D), jnp.float32),
            pltpu.VMEM_SHARED((_NP,), jnp.float32),
            pltpu.SemaphoreType.DMA((2,)),
        ],
    )(h, srcg, dstg, pq[:, 0], pq[:, 1], b)

    st = s2.T
    out = pl.pallas_call(
        _div_body,
        grid=(_NP // _RB,),
        in_specs=[pl.BlockSpec((2, _RB, _D), lambda i: (0, i, 0)),
                  pl.BlockSpec((_RB, 2), lambda i: (i, 0))],
        out_specs=pl.BlockSpec((_RB, _D), lambda i: (i, 0)),
        out_shape=jax.ShapeDtypeStruct((_NP, _D), jnp.float32),
    )(u2, st)
    return out[:_N]


# speedup vs baseline: 35.6573x; 1.0880x over previous
import jax
import jax.numpy as jnp
from jax import lax
from jax.experimental import pallas as pl
from jax.experimental.pallas import tpu as pltpu
from jax.experimental.pallas import tpu_sc as plsc

_N = 10000
_E = 320000
_D = 128
_NP = 10240
_NT = 32
_ET = _E // _NT
_K = 16
_KP = 16
_EC = 2000
_NC = _ET // _EC
_GC = _EC // _K
_NB = 8
_CW = _GC * _KP
_RB = 1280


def _pq_body(h_ref, w_ref, pq_ref, b_ref):
    h = h_ref[...]
    w = w_ref[...]
    pq = lax.dot_general(h, w, (((1,), (1,)), ((), ())),
                         preferred_element_type=jnp.float32)
    pq_ref[...] = pq
    s = jnp.max(pq[:, 0]) + jnp.max(pq[:, 1])
    b = jnp.where(s >= 0, s, 0.01 * s)
    b_ref[...] = jnp.full((8, 128), b, jnp.float32)


def _sc_body(h_hbm, srcg_hbm, dstg_hbm, p_hbm, q_hbm, b_hbm,
             u_hbm, s_hbm,
             src_c, dst_c, ex_c, p_v, q_v, b_v, rows8, szbuf,
             u_sh, s_sh, gsem, ssem):
    cid = lax.axis_index("c")
    sid = lax.axis_index("s")
    wid = cid * 16 + sid
    nbase = sid * 640

    zv = jnp.zeros((16,), jnp.float32)

    def zrow(r, c):
        for c8 in range(8):
            rows8[0, r, pl.ds(16 * c8, 16)] = zv
        return c
    lax.fori_loop(0, 16, zrow, 0)

    def zs(i, c):
        szbuf[pl.ds(16 * i, 16)] = zv
        return c
    lax.fori_loop(0, 40, zs, 0)

    for k in range(40):
        pltpu.sync_copy(rows8.at[0], u_sh.at[pl.ds(nbase + 16 * k, 16)])
    pltpu.sync_copy(szbuf, s_sh.at[pl.ds(nbase, 640)])

    pltpu.sync_copy(p_hbm, p_v)
    pltpu.sync_copy(q_hbm, q_v)
    pltpu.sync_copy(b_hbm.at[0], b_v)

    plsc.subcore_barrier()

    iota = lax.iota(jnp.int32, 16)
    bvec = b_v[pl.ds(0, 16)]

    def chunk(ci, cc):
        w = wid * _NC + ci
        pltpu.sync_copy(srcg_hbm.at[w, 0], src_c)
        pltpu.sync_copy(dstg_hbm.at[w, 0], dst_c)

        for b in range(4):
            pltpu.async_copy(h_hbm.at[src_c.at[pl.ds(16 * b, 16)]],
                             rows8.at[b], gsem.at[b])

        def exg(t, c):
            sv = src_c[pl.ds(16 * t, 16)]
            dv = dst_c[pl.ds(16 * t, 16)]
            pv = plsc.load_gather(p_v, [sv])
            qv = plsc.load_gather(q_v, [dv])
            a = pv + qv
            e = jnp.where(a >= 0, a, a * 0.01)
            ex_c[pl.ds(16 * t, 16)] = jnp.exp(e - bvec)
            return c
        lax.fori_loop(0, _GC, exg, 0)

        def heavy(g, c):
            eb = g * _KP
            b = jnp.bitwise_and(g, _NB - 1)
            bf = jnp.full((16,), b, jnp.int32)

            @pl.when(g + 4 < _GC)
            def _():
                b4 = jnp.bitwise_and(g + 4, _NB - 1)
                eb4 = (g + 4) * _KP

                @pl.when(g >= 4)
                def _():
                    ebm = (g - 4) * _KP
                    pltpu.make_async_copy(
                        rows8.at[b4],
                        u_sh.at[dst_c.at[pl.ds(ebm, _K)]],
                        ssem.at[b4]).wait()
                    pltpu.make_async_copy(
                        ex_c.at[pl.ds(ebm, _K)],
                        s_sh.at[dst_c.at[pl.ds(ebm, _K)]],
                        ssem.at[b4]).wait()
                pltpu.async_copy(h_hbm.at[src_c.at[pl.ds(eb4, _K)]],
                                 rows8.at[b4], gsem.at[b4])

            pltpu.make_async_copy(h_hbm.at[src_c.at[pl.ds(eb, _K)]],
                                  rows8.at[b], gsem.at[b]).wait()

            exv16 = ex_c[pl.ds(eb, 16)]

            @plsc.parallel_loop(0, 16, unroll=2)
            def scale_row(i):
                exv = exv16[jnp.full((16,), i, jnp.int32)]
                jf = jnp.full((16,), i, jnp.int32)
                for c8 in range(8):
                    cvec = iota + 16 * c8
                    x = plsc.load_gather(rows8, [bf, jf, cvec])
                    plsc.store_scatter(rows8, [bf, jf, cvec], x * exv)

            pltpu.async_copy(rows8.at[b], u_sh.at[dst_c.at[pl.ds(eb, _K)]],
                             ssem.at[b], add=True)
            pltpu.async_copy(ex_c.at[pl.ds(eb, _K)],
                             s_sh.at[dst_c.at[pl.ds(eb, _K)]],
                             ssem.at[b], add=True)
            return c
        lax.fori_loop(0, _GC, heavy, 0)

        for k in range(_NB):
            grp = _GC - _NB + k
            b = grp % _NB
            eb = grp * _KP
            pltpu.make_async_copy(rows8.at[b],
                                  u_sh.at[dst_c.at[pl.ds(eb, _K)]],
                                  ssem.at[b]).wait()
            pltpu.make_async_copy(ex_c.at[pl.ds(eb, _K)],
                                  s_sh.at[dst_c.at[pl.ds(eb, _K)]],
                                  ssem.at[b]).wait()
        return cc
    lax.fori_loop(0, _NC, chunk, 0)

    plsc.subcore_barrier()

    for k in range(40):
        pltpu.sync_copy(u_sh.at[pl.ds(nbase + 16 * k, 16)], rows8.at[0])
        pltpu.sync_copy(rows8.at[0], u_hbm.at[cid, pl.ds(nbase + 16 * k, 16)])
    pltpu.sync_copy(s_sh.at[pl.ds(nbase, 640)], szbuf)
    pltpu.sync_copy(szbuf, s_hbm.at[cid, pl.ds(nbase, 640)])


def _div_body(u_ref, st_ref, o_ref):
    num = u_ref[0] + u_ref[1]
    st = st_ref[...]
    den = st[:, 0:1] + st[:, 1:2]
    o_ref[...] = jnp.where(den > 0, num / den, 0.0)


def _pad_groups(x):
    xg = x.reshape(_NT * _NC, _GC, _K)
    xg = jnp.pad(xg, ((0, 0), (0, 0), (0, _KP - _K)))
    return xg.reshape(_NT * _NC, 1, _CW)


@jax.jit
def kernel(h, edge_index, attn_w):
    w = attn_w[:, 0].reshape(2, _D)

    pq, b = pl.pallas_call(
        _pq_body,
        out_shape=[jax.ShapeDtypeStruct((_N, 2), jnp.float32),
                   jax.ShapeDtypeStruct((8, 128), jnp.float32)],
    )(h, w)

    srcg = _pad_groups(edge_index[0])
    dstg = _pad_groups(edge_index[1])

    mesh = plsc.VectorSubcoreMesh(core_axis_name="c", subcore_axis_name="s",
                                  num_cores=2)
    u2, s2 = pl.kernel(
        _sc_body,
        out_type=[jax.ShapeDtypeStruct((2, _NP, _D), jnp.float32),
                  jax.ShapeDtypeStruct((2, _NP), jnp.float32)],
        mesh=mesh,
        compiler_params=pltpu.CompilerParams(needs_layout_passes=False),
        scratch_types=[
            pltpu.VMEM((_CW,), jnp.int32),
            pltpu.VMEM((_CW,), jnp.int32),
            pltpu.VMEM((_CW,), jnp.float32),
            pltpu.VMEM((_N,), jnp.float32),
            pltpu.VMEM((_N,), jnp.float32),
            pltpu.VMEM((128,), jnp.float32),
            pltpu.VMEM((_NB, _K, _D), jnp.float32),
            pltpu.VMEM((640,), jnp.float32),
            pltpu.VMEM_SHARED((_NP, _D), jnp.float32),
            pltpu.VMEM_SHARED((_NP,), jnp.float32),
            pltpu.SemaphoreType.DMA((_NB,)),
            pltpu.SemaphoreType.DMA((_NB,)),
        ],
    )(h, srcg, dstg, pq[:, 0], pq[:, 1], b)

    st = s2.T
    out = pl.pallas_call(
        _div_body,
        grid=(_NP // _RB,),
        in_specs=[pl.BlockSpec((2, _RB, _D), lambda i: (0, i, 0)),
                  pl.BlockSpec((_RB, 2), lambda i: (i, 0))],
        out_specs=pl.BlockSpec((_RB, _D), lambda i: (i, 0)),
        out_shape=jax.ShapeDtypeStruct((_NP, _D), jnp.float32),
    )(u2, st)
    return out[:_N]


# speedup vs baseline: 36.7393x; 1.0303x over previous
import jax
import jax.numpy as jnp
from jax import lax
from jax.experimental import pallas as pl
from jax.experimental.pallas import tpu as pltpu
from jax.experimental.pallas import tpu_sc as plsc

_N = 10000
_E = 320000
_D = 128
_NP = 10240
_NT = 32
_ET = _E // _NT
_K = 16
_KP = 16
_EC = 2000
_NC = _ET // _EC
_GC = _EC // _K
_NB = 8
_CW = _GC * _KP
_RB = 1000


def _pq_body(h_ref, w_ref, pq_ref, b_ref):
    h = h_ref[...]
    w = w_ref[...]
    pq = lax.dot_general(h, w, (((1,), (1,)), ((), ())),
                         preferred_element_type=jnp.float32)
    pq_ref[...] = pq
    s = jnp.max(pq[:, 0]) + jnp.max(pq[:, 1])
    b = jnp.where(s >= 0, s, 0.01 * s)
    b_ref[...] = jnp.full((8, 128), b, jnp.float32)


def _sc_body(h_hbm, srcg_hbm, dstg_hbm, p_hbm, q_hbm, b_hbm,
             u_hbm, s_hbm,
             src_c, dst_c, ex_c, p_v, q_v, b_v, rows8, szbuf,
             u_sh, s_sh, gsem, ssem):
    cid = lax.axis_index("c")
    sid = lax.axis_index("s")
    wid = cid * 16 + sid
    nbase = sid * 640

    zv = jnp.zeros((16,), jnp.float32)

    def zrow(r, c):
        for c8 in range(8):
            rows8[0, r, pl.ds(16 * c8, 16)] = zv
        return c
    lax.fori_loop(0, 16, zrow, 0)

    def zs(i, c):
        szbuf[pl.ds(16 * i, 16)] = zv
        return c
    lax.fori_loop(0, 40, zs, 0)

    for k in range(40):
        pltpu.sync_copy(rows8.at[0], u_sh.at[pl.ds(nbase + 16 * k, 16)])
    pltpu.sync_copy(szbuf, s_sh.at[pl.ds(nbase, 640)])

    pltpu.sync_copy(p_hbm, p_v)
    pltpu.sync_copy(q_hbm, q_v)
    pltpu.sync_copy(b_hbm.at[0], b_v)

    plsc.subcore_barrier()

    iota = lax.iota(jnp.int32, 16)
    bvec = b_v[pl.ds(0, 16)]

    def chunk(ci, cc):
        ebase = wid * _ET + ci * _EC
        pltpu.sync_copy(srcg_hbm.at[pl.ds(ebase, _EC)], src_c)
        pltpu.sync_copy(dstg_hbm.at[pl.ds(ebase, _EC)], dst_c)

        for b in range(4):
            pltpu.async_copy(h_hbm.at[src_c.at[pl.ds(16 * b, 16)]],
                             rows8.at[b], gsem.at[b])

        def exg(t, c):
            sv = src_c[pl.ds(16 * t, 16)]
            dv = dst_c[pl.ds(16 * t, 16)]
            pv = plsc.load_gather(p_v, [sv])
            qv = plsc.load_gather(q_v, [dv])
            a = pv + qv
            e = jnp.where(a >= 0, a, a * 0.01)
            ex_c[pl.ds(16 * t, 16)] = jnp.exp(e - bvec)
            return c
        lax.fori_loop(0, _GC, exg, 0)

        def heavy(g, c):
            eb = g * _KP
            b = jnp.bitwise_and(g, _NB - 1)
            bf = jnp.full((16,), b, jnp.int32)

            @pl.when(g + 4 < _GC)
            def _():
                b4 = jnp.bitwise_and(g + 4, _NB - 1)
                eb4 = (g + 4) * _KP

                @pl.when(g >= 4)
                def _():
                    ebm = (g - 4) * _KP
                    pltpu.make_async_copy(
                        rows8.at[b4],
                        u_sh.at[dst_c.at[pl.ds(ebm, _K)]],
                        ssem.at[b4]).wait()
                    pltpu.make_async_copy(
                        ex_c.at[pl.ds(ebm, _K)],
                        s_sh.at[dst_c.at[pl.ds(ebm, _K)]],
                        ssem.at[b4]).wait()
                pltpu.async_copy(h_hbm.at[src_c.at[pl.ds(eb4, _K)]],
                                 rows8.at[b4], gsem.at[b4])

            pltpu.make_async_copy(h_hbm.at[src_c.at[pl.ds(eb, _K)]],
                                  rows8.at[b], gsem.at[b]).wait()

            exv16 = ex_c[pl.ds(eb, 16)]

            @plsc.parallel_loop(0, 16, unroll=2)
            def scale_row(i):
                exv = exv16[jnp.full((16,), i, jnp.int32)]
                jf = jnp.full((16,), i, jnp.int32)
                for c8 in range(8):
                    cvec = iota + 16 * c8
                    x = plsc.load_gather(rows8, [bf, jf, cvec])
                    plsc.store_scatter(rows8, [bf, jf, cvec], x * exv)

            pltpu.async_copy(rows8.at[b], u_sh.at[dst_c.at[pl.ds(eb, _K)]],
                             ssem.at[b], add=True)
            pltpu.async_copy(ex_c.at[pl.ds(eb, _K)],
                             s_sh.at[dst_c.at[pl.ds(eb, _K)]],
                             ssem.at[b], add=True)
            return c
        lax.fori_loop(0, _GC, heavy, 0)

        for k in range(_NB):
            grp = _GC - _NB + k
            b = grp % _NB
            eb = grp * _KP
            pltpu.make_async_copy(rows8.at[b],
                                  u_sh.at[dst_c.at[pl.ds(eb, _K)]],
                                  ssem.at[b]).wait()
            pltpu.make_async_copy(ex_c.at[pl.ds(eb, _K)],
                                  s_sh.at[dst_c.at[pl.ds(eb, _K)]],
                                  ssem.at[b]).wait()
        return cc
    lax.fori_loop(0, _NC, chunk, 0)

    plsc.subcore_barrier()

    for k in range(40):
        pltpu.sync_copy(u_sh.at[pl.ds(nbase + 16 * k, 16)], rows8.at[0])
        pltpu.sync_copy(rows8.at[0], u_hbm.at[cid, pl.ds(nbase + 16 * k, 16)])
    pltpu.sync_copy(s_sh.at[pl.ds(nbase, 640)], szbuf)
    pltpu.sync_copy(szbuf, s_hbm.at[cid, pl.ds(nbase, 640)])


def _div_body(u_ref, st_ref, o_ref):
    num = u_ref[0] + u_ref[1]
    st = st_ref[...]
    den = st[:, 0:1] + st[:, 1:2]
    o_ref[...] = jnp.where(den > 0, num / den, 0.0)


@jax.jit
def kernel(h, edge_index, attn_w):
    w = attn_w[:, 0].reshape(2, _D)

    pq, b = pl.pallas_call(
        _pq_body,
        out_shape=[jax.ShapeDtypeStruct((_N, 2), jnp.float32),
                   jax.ShapeDtypeStruct((8, 128), jnp.float32)],
    )(h, w)

    srcg = edge_index[0]
    dstg = edge_index[1]

    mesh = plsc.VectorSubcoreMesh(core_axis_name="c", subcore_axis_name="s",
                                  num_cores=2)
    u2, s2 = pl.kernel(
        _sc_body,
        out_type=[jax.ShapeDtypeStruct((2, _NP, _D), jnp.float32),
                  jax.ShapeDtypeStruct((2, _NP), jnp.float32)],
        mesh=mesh,
        compiler_params=pltpu.CompilerParams(needs_layout_passes=False),
        scratch_types=[
            pltpu.VMEM((_CW,), jnp.int32),
            pltpu.VMEM((_CW,), jnp.int32),
            pltpu.VMEM((_CW,), jnp.float32),
            pltpu.VMEM((_N,), jnp.float32),
            pltpu.VMEM((_N,), jnp.float32),
            pltpu.VMEM((128,), jnp.float32),
            pltpu.VMEM((_NB, _K, _D), jnp.float32),
            pltpu.VMEM((640,), jnp.float32),
            pltpu.VMEM_SHARED((_NP, _D), jnp.float32),
            pltpu.VMEM_SHARED((_NP,), jnp.float32),
            pltpu.SemaphoreType.DMA((_NB,)),
            pltpu.SemaphoreType.DMA((_NB,)),
        ],
    )(h, srcg, dstg, pq[:, 0], pq[:, 1], b)

    st = s2.T
    out = pl.pallas_call(
        _div_body,
        grid=(_N // _RB,),
        in_specs=[pl.BlockSpec((2, _RB, _D), lambda i: (0, i, 0)),
                  pl.BlockSpec((_RB, 2), lambda i: (i, 0))],
        out_specs=pl.BlockSpec((_RB, _D), lambda i: (i, 0)),
        out_shape=jax.ShapeDtypeStruct((_N, _D), jnp.float32),
    )(u2, st)
    return out
